# baseline (speedup 1.0000x reference)
import functools

import jax
import jax.numpy as jnp
from jax.experimental import pallas as pl
from jax.experimental.pallas import tpu as pltpu

_IB = 32


def _loss_block(gt_ref, pr_ref, out_ref, s_ref, c_ref):
    i = pl.program_id(0)

    @pl.when(i == 0)
    def _init():
        s_ref[0] = 0.0
        c_ref[0] = 0.0

    g = gt_ref[...]
    p = pr_ref[...]
    ib, h, w = g.shape
    m2v = jnp.maximum(g[:, :-1, :], g[:, 1:, :])
    v3 = jnp.maximum(m2v[:, :-1, :], m2v[:, 1:, :])
    vp = jnp.pad(v3, ((0, 0), (1, 1), (0, 256 - w)),
                 constant_values=jnp.inf)
    wm = jnp.maximum(pltpu.roll(vp, 1, 2),
                     jnp.maximum(vp, pltpu.roll(vp, 255, 2)))[:, :, :w]
    mask = (wm == g) & (g > 0.0)
    s_ref[0] += jnp.sum(jnp.where(mask, jnp.abs(p - g), 0.0))
    c_ref[0] += jnp.sum(jnp.where(mask, 1.0, 0.0))

    @pl.when(i == pl.num_programs(0) - 1)
    def _fin():
        out_ref[0] = s_ref[0] / (c_ref[0] + 0.0001)


def kernel(predict, gt):
    n = gt.shape[0] * gt.shape[1]
    h, w = gt.shape[2], gt.shape[3]
    g3 = gt.reshape(n, h, w)
    p3 = predict.reshape(n, h, w)
    grid = (n // _IB,)
    loss = pl.pallas_call(
        _loss_block,
        grid=grid,
        in_specs=[
            pl.BlockSpec((_IB, h, w), lambda i: (i, 0, 0)),
            pl.BlockSpec((_IB, h, w), lambda i: (i, 0, 0)),
        ],
        out_specs=pl.BlockSpec(memory_space=pltpu.SMEM),
        out_shape=jax.ShapeDtypeStruct((1,), jnp.float32),
        scratch_shapes=[
            pltpu.SMEM((1,), jnp.float32),
            pltpu.SMEM((1,), jnp.float32),
        ],
    )(g3, p3)
    return loss[0]
